# baseline (speedup 1.0000x reference)
import functools

import jax
import jax.numpy as jnp
from jax.experimental import pallas as pl


def _pre_body(ent_ref, gw_ref, asrc_ref, adst_ref, hx_ref, src_ref, ed_ref,
              ed2_ref, dmax_ref):
    ent = ent_ref[...]
    hx = jnp.dot(ent.astype(jnp.bfloat16), gw_ref[...],
                 preferred_element_type=jnp.float32)
    hxb = hx.astype(jnp.bfloat16)
    n = hx.shape[0]
    hx_ref[...] = jnp.concatenate(
        [hxb, jnp.ones((n, 1), jnp.bfloat16)], axis=1)
    src_ref[...] = jnp.dot(hxb, asrc_ref[...],
                           preferred_element_type=jnp.float32)
    dst = jax.lax.dot_general(
        adst_ref[...], hxb, (((1,), (1,)), ((), ())),
        preferred_element_type=jnp.float32)
    ed_ref[...] = jnp.exp(dst)
    ed2_ref[...] = jnp.exp(0.2 * dst)
    dmax_ref[...] = jnp.max(dst, axis=1, keepdims=True)


def _main_body(adj_ref, ent_ref, src_ref, rel_ref, hx_ref, ed_ref,
               ed2_ref, dmax_ref, relemb_ref, wq_ref, wk_ref, wv_ref,
               out_ref, *, d):
    adj = adj_ref[...]
    src = src_ref[...]
    mt = jnp.maximum(src + dmax_ref[...], 0.0)
    sa = jnp.exp(src - mt)
    sb = jnp.exp(0.2 * src - mt)
    p = jnp.maximum(sa * ed_ref[...], sb * ed2_ref[...]) * adj
    o_ext = jnp.dot(p.astype(jnp.bfloat16), hx_ref[...],
                    preferred_element_type=jnp.float32)
    h = o_ext[:, :d] / o_ext[:, d:d + 1]
    nrm = jnp.sqrt(jnp.sum(h * h, axis=1, keepdims=True))
    hn = h / jnp.maximum(nrm, 1e-12)

    entb = ent_ref[...].astype(jnp.bfloat16)
    hnb = hn.astype(jnp.bfloat16)
    wq = wq_ref[...]
    wk = wk_ref[...]
    wv = wv_ref[...]
    q0 = jnp.dot(entb, wq, preferred_element_type=jnp.float32)
    q1 = jnp.dot(hnb, wq, preferred_element_type=jnp.float32)
    k0 = jnp.dot(entb, wk, preferred_element_type=jnp.float32)
    k1 = jnp.dot(hnb, wk, preferred_element_type=jnp.float32)
    v0 = jnp.dot(entb, wv, preferred_element_type=jnp.float32)
    v1 = jnp.dot(hnb, wv, preferred_element_type=jnp.float32)
    scale = 1.0 / jnp.sqrt(jnp.float32(d))
    s00 = jnp.sum(q0 * k0, axis=1, keepdims=True) * scale
    s01 = jnp.sum(q0 * k1, axis=1, keepdims=True) * scale
    s10 = jnp.sum(q1 * k0, axis=1, keepdims=True) * scale
    s11 = jnp.sum(q1 * k1, axis=1, keepdims=True) * scale
    m0 = jnp.maximum(s00, s01)
    m1 = jnp.maximum(s10, s11)
    e00 = jnp.exp(s00 - m0)
    e01 = jnp.exp(s01 - m0)
    e10 = jnp.exp(s10 - m1)
    e11 = jnp.exp(s11 - m1)
    o0 = (e00 * v0 + e01 * v1) / (e00 + e01)
    o1 = (e10 * v0 + e11 * v1) / (e10 + e11)
    emb = 0.5 * (o0 + o1)

    ra = rel_ref[...]
    rs = jnp.sum(ra, axis=1, keepdims=True)
    ro = jnp.dot(ra.astype(jnp.bfloat16), relemb_ref[...],
                 preferred_element_type=jnp.float32) / rs

    out_ref[...] = jnp.concatenate([emb, ro], axis=1)


def _branch(adj, rel_adj, ent, rel_emb_b, gw_b, asrc_b, adst_b,
            wq_b, wk_b, wv_b, block_rows):
    n, d = ent.shape
    r = rel_adj.shape[1]
    b = block_rows

    hx, src, ed, ed2, dmax = pl.pallas_call(
        _pre_body,
        out_shape=(
            jax.ShapeDtypeStruct((n, d + 1), jnp.bfloat16),
            jax.ShapeDtypeStruct((n, 1), jnp.float32),
            jax.ShapeDtypeStruct((1, n), jnp.float32),
            jax.ShapeDtypeStruct((1, n), jnp.float32),
            jax.ShapeDtypeStruct((1, 1), jnp.float32),
        ),
    )(ent, gw_b, asrc_b, adst_b)

    body = functools.partial(_main_body, d=d)
    out = pl.pallas_call(
        body,
        grid=(n // b,),
        in_specs=[
            pl.BlockSpec((b, n), lambda i: (i, 0)),
            pl.BlockSpec((b, d), lambda i: (i, 0)),
            pl.BlockSpec((b, 1), lambda i: (i, 0)),
            pl.BlockSpec((b, r), lambda i: (i, 0)),
            pl.BlockSpec((n, d + 1), lambda i: (0, 0)),
            pl.BlockSpec((1, n), lambda i: (0, 0)),
            pl.BlockSpec((1, n), lambda i: (0, 0)),
            pl.BlockSpec((1, 1), lambda i: (0, 0)),
            pl.BlockSpec((r, d), lambda i: (0, 0)),
            pl.BlockSpec((d, d), lambda i: (0, 0)),
            pl.BlockSpec((d, d), lambda i: (0, 0)),
            pl.BlockSpec((d, d), lambda i: (0, 0)),
        ],
        out_specs=pl.BlockSpec((b, 2 * d), lambda i: (i, 0)),
        out_shape=jax.ShapeDtypeStruct((n, 2 * d), jnp.float32),
    )(adj, ent, src, rel_adj, hx, ed, ed2, dmax, rel_emb_b, wq_b, wk_b, wv_b)
    return out


def kernel(adj_sr, adj_tg, rel_adj_sr, rel_adj_tg, ent_sr, ent_tg,
           rel_sr, rel_tg, gat_W, gat_a_src, gat_a_dst, Wq, Wk, Wv):
    n, d = ent_sr.shape
    block_rows = 80 if n % 80 == 0 else n

    gw_b = gat_W.astype(jnp.bfloat16)
    asrc_b = gat_a_src.reshape(d, 1).astype(jnp.bfloat16)
    adst_b = gat_a_dst.reshape(1, d).astype(jnp.bfloat16)
    wq_b = Wq.astype(jnp.bfloat16)
    wk_b = Wk.astype(jnp.bfloat16)
    wv_b = Wv.astype(jnp.bfloat16)

    sr_out = _branch(adj_sr, rel_adj_sr, ent_sr, rel_sr.astype(jnp.bfloat16),
                     gw_b, asrc_b, adst_b, wq_b, wk_b, wv_b, block_rows)
    tg_out = _branch(adj_tg, rel_adj_tg, ent_tg, rel_tg.astype(jnp.bfloat16),
                     gw_b, asrc_b, adst_b, wq_b, wk_b, wv_b, block_rows)
    return (sr_out, tg_out)


# speedup vs baseline: 1.4905x; 1.4905x over previous
import functools

import jax
import jax.numpy as jnp
from jax.experimental import pallas as pl


def _pre_body(ent_ref, gw_ref, asrc_ref, adst_ref, hx_ref, src_ref, ed_ref,
              ed2_ref, dmax_ref):
    ent = ent_ref[...]
    hx = jnp.dot(ent.astype(jnp.bfloat16), gw_ref[...],
                 preferred_element_type=jnp.float32)
    hxb = hx.astype(jnp.bfloat16)
    n = hx.shape[0]
    hx_ref[...] = jnp.concatenate(
        [hxb, jnp.ones((n, 1), jnp.bfloat16)], axis=1)
    src_ref[...] = jnp.dot(hxb, asrc_ref[...],
                           preferred_element_type=jnp.float32)
    dst = jax.lax.dot_general(
        adst_ref[...], hxb, (((1,), (1,)), ((), ())),
        preferred_element_type=jnp.float32)
    ed_ref[...] = jnp.exp(dst)
    ed2_ref[...] = jnp.exp(0.2 * dst)
    dmax_ref[...] = jnp.max(dst, axis=1, keepdims=True)


def _main_body(adj_ref, ent_ref, src_ref, rel_ref, hx_ref, ed_ref,
               ed2_ref, dmax_ref, relemb_ref, wq_ref, wk_ref, wv_ref,
               out_ref, *, d):
    adj = adj_ref[...]
    src = src_ref[...]
    mt = jnp.maximum(src + dmax_ref[...], 0.0)
    sa = jnp.exp(src - mt)
    sb = jnp.exp(0.2 * src - mt)
    p = jnp.maximum(sa * ed_ref[...], sb * ed2_ref[...]) * adj
    o_ext = jnp.dot(p.astype(jnp.bfloat16), hx_ref[...],
                    preferred_element_type=jnp.float32)
    h = o_ext[:, :d] / o_ext[:, d:d + 1]
    nrm = jnp.sqrt(jnp.sum(h * h, axis=1, keepdims=True))
    hn = h / jnp.maximum(nrm, 1e-12)

    entb = ent_ref[...].astype(jnp.bfloat16)
    hnb = hn.astype(jnp.bfloat16)
    wq = wq_ref[...]
    wk = wk_ref[...]
    wv = wv_ref[...]
    q0 = jnp.dot(entb, wq, preferred_element_type=jnp.float32)
    q1 = jnp.dot(hnb, wq, preferred_element_type=jnp.float32)
    k0 = jnp.dot(entb, wk, preferred_element_type=jnp.float32)
    k1 = jnp.dot(hnb, wk, preferred_element_type=jnp.float32)
    v0 = jnp.dot(entb, wv, preferred_element_type=jnp.float32)
    v1 = jnp.dot(hnb, wv, preferred_element_type=jnp.float32)
    scale = 1.0 / jnp.sqrt(jnp.float32(d))
    s00 = jnp.sum(q0 * k0, axis=1, keepdims=True) * scale
    s01 = jnp.sum(q0 * k1, axis=1, keepdims=True) * scale
    s10 = jnp.sum(q1 * k0, axis=1, keepdims=True) * scale
    s11 = jnp.sum(q1 * k1, axis=1, keepdims=True) * scale
    m0 = jnp.maximum(s00, s01)
    m1 = jnp.maximum(s10, s11)
    e00 = jnp.exp(s00 - m0)
    e01 = jnp.exp(s01 - m0)
    e10 = jnp.exp(s10 - m1)
    e11 = jnp.exp(s11 - m1)
    o0 = (e00 * v0 + e01 * v1) / (e00 + e01)
    o1 = (e10 * v0 + e11 * v1) / (e10 + e11)
    emb = 0.5 * (o0 + o1)

    ra = rel_ref[...]
    rs = jnp.sum(ra, axis=1, keepdims=True)
    ro = jnp.dot(ra.astype(jnp.bfloat16), relemb_ref[...],
                 preferred_element_type=jnp.float32) / rs

    out_ref[...] = jnp.concatenate([emb, ro], axis=1)


def _branch(adj, rel_adj, ent, rel_emb_b, gw_b, asrc_b, adst_b,
            wq_b, wk_b, wv_b, block_rows):
    n, d = ent.shape
    r = rel_adj.shape[1]
    b = block_rows

    hx, src, ed, ed2, dmax = pl.pallas_call(
        _pre_body,
        out_shape=(
            jax.ShapeDtypeStruct((n, d + 1), jnp.bfloat16),
            jax.ShapeDtypeStruct((n, 1), jnp.float32),
            jax.ShapeDtypeStruct((1, n), jnp.float32),
            jax.ShapeDtypeStruct((1, n), jnp.float32),
            jax.ShapeDtypeStruct((1, 1), jnp.float32),
        ),
    )(ent, gw_b, asrc_b, adst_b)

    body = functools.partial(_main_body, d=d)
    out = pl.pallas_call(
        body,
        grid=(n // b,),
        in_specs=[
            pl.BlockSpec((b, n), lambda i: (i, 0)),
            pl.BlockSpec((b, d), lambda i: (i, 0)),
            pl.BlockSpec((b, 1), lambda i: (i, 0)),
            pl.BlockSpec((b, r), lambda i: (i, 0)),
            pl.BlockSpec((n, d + 1), lambda i: (0, 0)),
            pl.BlockSpec((1, n), lambda i: (0, 0)),
            pl.BlockSpec((1, n), lambda i: (0, 0)),
            pl.BlockSpec((1, 1), lambda i: (0, 0)),
            pl.BlockSpec((r, d), lambda i: (0, 0)),
            pl.BlockSpec((d, d), lambda i: (0, 0)),
            pl.BlockSpec((d, d), lambda i: (0, 0)),
            pl.BlockSpec((d, d), lambda i: (0, 0)),
        ],
        out_specs=pl.BlockSpec((b, 2 * d), lambda i: (i, 0)),
        out_shape=jax.ShapeDtypeStruct((n, 2 * d), jnp.float32),
    )(adj, ent, src, rel_adj, hx, ed, ed2, dmax, rel_emb_b, wq_b, wk_b, wv_b)
    return out


def kernel(adj_sr, adj_tg, rel_adj_sr, rel_adj_tg, ent_sr, ent_tg,
           rel_sr, rel_tg, gat_W, gat_a_src, gat_a_dst, Wq, Wk, Wv):
    n, d = ent_sr.shape
    block_rows = 400 if n % 400 == 0 else n

    gw_b = gat_W.astype(jnp.bfloat16)
    asrc_b = gat_a_src.reshape(d, 1).astype(jnp.bfloat16)
    adst_b = gat_a_dst.reshape(1, d).astype(jnp.bfloat16)
    wq_b = Wq.astype(jnp.bfloat16)
    wk_b = Wk.astype(jnp.bfloat16)
    wv_b = Wv.astype(jnp.bfloat16)

    sr_out = _branch(adj_sr, rel_adj_sr, ent_sr, rel_sr.astype(jnp.bfloat16),
                     gw_b, asrc_b, adst_b, wq_b, wk_b, wv_b, block_rows)
    tg_out = _branch(adj_tg, rel_adj_tg, ent_tg, rel_tg.astype(jnp.bfloat16),
                     gw_b, asrc_b, adst_b, wq_b, wk_b, wv_b, block_rows)
    return (sr_out, tg_out)


# speedup vs baseline: 1.4960x; 1.0036x over previous
import functools

import jax
import jax.numpy as jnp
from jax.experimental import pallas as pl


def _pre_body(ent_ref, gw_ref, asrc_ref, adst_ref, hx_ref, src_ref, ed_ref,
              ed2_ref, dmax_ref):
    ent = ent_ref[...]
    hx = jnp.dot(ent.astype(jnp.bfloat16), gw_ref[...],
                 preferred_element_type=jnp.float32)
    hxb = hx.astype(jnp.bfloat16)
    n = hx.shape[0]
    hx_ref[...] = jnp.concatenate(
        [hxb, jnp.ones((n, 1), jnp.bfloat16)], axis=1)
    src_ref[...] = jnp.dot(hxb, asrc_ref[...],
                           preferred_element_type=jnp.float32)
    dst = jax.lax.dot_general(
        adst_ref[...], hxb, (((1,), (1,)), ((), ())),
        preferred_element_type=jnp.float32)
    ed_ref[...] = jnp.exp(dst)
    ed2_ref[...] = jnp.exp(0.2 * dst)
    dmax_ref[...] = jnp.max(dst, axis=1, keepdims=True)


def _main_body(adjl_ref, adjr_ref, ent_ref, src_ref, rel_ref, hx_ref, ed_ref,
               ed2_ref, dmax_ref, relemb_ref, wq_ref, wk_ref, wv_ref,
               out_ref, *, d):
    src = src_ref[...]
    mt = jnp.maximum(src + dmax_ref[...], 0.0)
    sa = jnp.exp(src - mt)
    sb = jnp.exp(0.2 * src - mt)
    hb = src.shape[0] // 2
    ed = ed_ref[...]
    ed2 = ed2_ref[...]
    hx = hx_ref[...]
    pt = jnp.maximum(sa[:hb] * ed, sb[:hb] * ed2) * adjl_ref[...]
    pb = jnp.maximum(sa[hb:] * ed, sb[hb:] * ed2) * adjr_ref[...]
    o_ext = jnp.concatenate(
        [jnp.dot(pt.astype(jnp.bfloat16), hx,
                 preferred_element_type=jnp.float32),
         jnp.dot(pb.astype(jnp.bfloat16), hx,
                 preferred_element_type=jnp.float32)], axis=0)
    h = o_ext[:, :d] / o_ext[:, d:d + 1]
    nrm = jnp.sqrt(jnp.sum(h * h, axis=1, keepdims=True))
    hn = h / jnp.maximum(nrm, 1e-12)

    entb = ent_ref[...].astype(jnp.bfloat16)
    hnb = hn.astype(jnp.bfloat16)
    wq = wq_ref[...]
    wk = wk_ref[...]
    wv = wv_ref[...]
    q0 = jnp.dot(entb, wq, preferred_element_type=jnp.float32)
    q1 = jnp.dot(hnb, wq, preferred_element_type=jnp.float32)
    k0 = jnp.dot(entb, wk, preferred_element_type=jnp.float32)
    k1 = jnp.dot(hnb, wk, preferred_element_type=jnp.float32)
    v0 = jnp.dot(entb, wv, preferred_element_type=jnp.float32)
    v1 = jnp.dot(hnb, wv, preferred_element_type=jnp.float32)
    scale = 1.0 / jnp.sqrt(jnp.float32(d))
    s00 = jnp.sum(q0 * k0, axis=1, keepdims=True) * scale
    s01 = jnp.sum(q0 * k1, axis=1, keepdims=True) * scale
    s10 = jnp.sum(q1 * k0, axis=1, keepdims=True) * scale
    s11 = jnp.sum(q1 * k1, axis=1, keepdims=True) * scale
    m0 = jnp.maximum(s00, s01)
    m1 = jnp.maximum(s10, s11)
    e00 = jnp.exp(s00 - m0)
    e01 = jnp.exp(s01 - m0)
    e10 = jnp.exp(s10 - m1)
    e11 = jnp.exp(s11 - m1)
    o0 = (e00 * v0 + e01 * v1) / (e00 + e01)
    o1 = (e10 * v0 + e11 * v1) / (e10 + e11)
    emb = 0.5 * (o0 + o1)

    ra = rel_ref[...]
    rs = jnp.sum(ra, axis=1, keepdims=True)
    ro = jnp.dot(ra.astype(jnp.bfloat16), relemb_ref[...],
                 preferred_element_type=jnp.float32) / rs

    out_ref[...] = jnp.concatenate([emb, ro], axis=1)


def _branch(adj, rel_adj, ent, rel_emb_b, gw_b, asrc_b, adst_b,
            wq_b, wk_b, wv_b, block_rows):
    n, d = ent.shape
    r = rel_adj.shape[1]
    b = block_rows

    hx, src, ed, ed2, dmax = pl.pallas_call(
        _pre_body,
        out_shape=(
            jax.ShapeDtypeStruct((n, d + 1), jnp.bfloat16),
            jax.ShapeDtypeStruct((n, 1), jnp.float32),
            jax.ShapeDtypeStruct((1, n), jnp.float32),
            jax.ShapeDtypeStruct((1, n), jnp.float32),
            jax.ShapeDtypeStruct((1, 1), jnp.float32),
        ),
    )(ent, gw_b, asrc_b, adst_b)

    body = functools.partial(_main_body, d=d)
    out = pl.pallas_call(
        body,
        grid=(n // b,),
        in_specs=[
            pl.BlockSpec((b // 2, n), lambda i: (2 * i, 0)),
            pl.BlockSpec((b // 2, n), lambda i: (2 * i + 1, 0)),
            pl.BlockSpec((b, d), lambda i: (i, 0)),
            pl.BlockSpec((b, 1), lambda i: (i, 0)),
            pl.BlockSpec((b, r), lambda i: (i, 0)),
            pl.BlockSpec((n, d + 1), lambda i: (0, 0)),
            pl.BlockSpec((1, n), lambda i: (0, 0)),
            pl.BlockSpec((1, n), lambda i: (0, 0)),
            pl.BlockSpec((1, 1), lambda i: (0, 0)),
            pl.BlockSpec((r, d), lambda i: (0, 0)),
            pl.BlockSpec((d, d), lambda i: (0, 0)),
            pl.BlockSpec((d, d), lambda i: (0, 0)),
            pl.BlockSpec((d, d), lambda i: (0, 0)),
        ],
        out_specs=pl.BlockSpec((b, 2 * d), lambda i: (i, 0)),
        out_shape=jax.ShapeDtypeStruct((n, 2 * d), jnp.float32),
    )(adj, adj, ent, src, rel_adj, hx, ed, ed2, dmax, rel_emb_b, wq_b, wk_b,
      wv_b)
    return out


def kernel(adj_sr, adj_tg, rel_adj_sr, rel_adj_tg, ent_sr, ent_tg,
           rel_sr, rel_tg, gat_W, gat_a_src, gat_a_dst, Wq, Wk, Wv):
    n, d = ent_sr.shape
    block_rows = 400 if n % 400 == 0 else n

    gw_b = gat_W.astype(jnp.bfloat16)
    asrc_b = gat_a_src.reshape(d, 1).astype(jnp.bfloat16)
    adst_b = gat_a_dst.reshape(1, d).astype(jnp.bfloat16)
    wq_b = Wq.astype(jnp.bfloat16)
    wk_b = Wk.astype(jnp.bfloat16)
    wv_b = Wv.astype(jnp.bfloat16)

    sr_out = _branch(adj_sr, rel_adj_sr, ent_sr, rel_sr.astype(jnp.bfloat16),
                     gw_b, asrc_b, adst_b, wq_b, wk_b, wv_b, block_rows)
    tg_out = _branch(adj_tg, rel_adj_tg, ent_tg, rel_tg.astype(jnp.bfloat16),
                     gw_b, asrc_b, adst_b, wq_b, wk_b, wv_b, block_rows)
    return (sr_out, tg_out)
